# baseline (speedup 1.0000x reference)
import jax
import jax.numpy as jnp
from jax import lax
from jax.experimental import pallas as pl
from jax.experimental.pallas import tpu as pltpu
from jax.experimental.pallas import tpu_sc as plsc

N = 10000
E = 320000
D = 128
G = 128
OUTC = 1

NC = 2
NS = 16
NW = NC * NS

CHUNK = 64
NCHUNK = 160
EPW = NCHUNK * CHUNK
EPAD = NW * EPW
NBUF = 4

N_ACC = 10112
ZROWS = N_ACC // NS

_mesh = plsc.VectorSubcoreMesh(
    core_axis_name="c", subcore_axis_name="s", num_cores=NC, num_subcores=NS)


def _mask_body(f_ref, m_ref, o_ref):
    o_ref[...] = f_ref[...] * m_ref[...]


_mask_call = pl.pallas_call(
    _mask_body, out_shape=jax.ShapeDtypeStruct((N, D), jnp.float32))


def _edge_agg_body(h_hbm, src_hbm, dst_hbm, z_hbm, out_hbm,
                   acc, src_v, dring, rowbuf, gsem, ssem, isem):
    c = lax.axis_index("c")
    s = lax.axis_index("s")
    wid = c * NS + s

    pltpu.sync_copy(src_hbm.at[wid], src_v)
    pltpu.sync_copy(z_hbm, acc.at[pl.ds(s * ZROWS, ZROWS)])
    plsc.subcore_barrier()

    def start_gather(j, b):
        pltpu.async_copy(h_hbm.at[src_v.at[pl.ds(j * CHUNK, CHUNK)]],
                         rowbuf.at[b], gsem.at[b])

    def wait_gather(j, b):
        pltpu.make_async_copy(h_hbm.at[src_v.at[pl.ds(j * CHUNK, CHUNK)]],
                              rowbuf.at[b], gsem.at[b]).wait()

    def start_idx(j, b):
        pltpu.async_copy(dst_hbm.at[wid, j], dring.at[b], isem.at[b])

    def wait_idx(j, b):
        pltpu.make_async_copy(dst_hbm.at[wid, j], dring.at[b],
                              isem.at[b]).wait()

    def start_scatter(b):
        pltpu.async_copy(rowbuf.at[b], acc.at[dring.at[b]], ssem.at[b],
                         add=True)

    def wait_scatter(b):
        pltpu.make_async_copy(rowbuf.at[b], acc.at[dring.at[b]],
                              ssem.at[b]).wait()

    for b in range(NBUF):
        start_gather(b, b)
        start_idx(b, b)

    def ring(g, carry):
        for b in range(NBUF):
            j = g * NBUF + b
            wait_gather(j, b)
            wait_idx(j, b)
            start_scatter(b)
            wait_scatter(b)
            start_gather(j + NBUF, b)
            start_idx(j + NBUF, b)
        return carry

    lax.fori_loop(0, NCHUNK // NBUF - 1, ring, 0)
    for b in range(NBUF):
        j = NCHUNK - NBUF + b
        wait_gather(j, b)
        wait_idx(j, b)
        start_scatter(b)
        wait_scatter(b)

    plsc.subcore_barrier()
    wbase = s * ZROWS
    pltpu.sync_copy(acc.at[pl.ds(wbase, ZROWS)],
                    out_hbm.at[c, pl.ds(wbase, ZROWS)])


_edge_agg_call = pl.kernel(
    _edge_agg_body,
    out_type=jax.ShapeDtypeStruct((NC, N_ACC, D), jnp.float32),
    mesh=_mesh,
    scratch_types=[
        pltpu.VMEM_SHARED((N_ACC, D), jnp.float32),
        pltpu.VMEM((EPW,), jnp.int32),
        pltpu.VMEM((NBUF, CHUNK), jnp.int32),
        pltpu.VMEM((NBUF, CHUNK, D), jnp.float32),
        pltpu.SemaphoreType.DMA((NBUF,)),
        pltpu.SemaphoreType.DMA((NBUF,)),
        pltpu.SemaphoreType.DMA((NBUF,)),
    ],
)


def _x1_body(x_ref, a_ref, w_ref, b_ref, o_ref):
    z = x_ref[...] + (a_ref[0, :N, :] + a_ref[1, :N, :])
    o_ref[...] = jnp.maximum(
        jnp.dot(z, w_ref[...], preferred_element_type=jnp.float32)
        + b_ref[...], 0.0)


_x1_call = pl.pallas_call(
    _x1_body, out_shape=jax.ShapeDtypeStruct((N, D), jnp.float32))


def _fin_body(x_ref, a_ref, w2_ref, b2_ref, wp_ref, bp_ref, bat_ref, o_ref):
    z = x_ref[...] + (a_ref[0, :N, :] + a_ref[1, :N, :])
    nr = jnp.dot(z, w2_ref[...],
                 preferred_element_type=jnp.float32) + b2_ref[...]
    gids = lax.broadcasted_iota(jnp.int32, (G, N), 0)
    onehot = (bat_ref[...] == gids).astype(jnp.float32)
    sums = jnp.dot(onehot, nr, preferred_element_type=jnp.float32,
                   precision=lax.Precision.HIGHEST)
    cnt = jnp.sum(onehot, axis=1, keepdims=True)
    pool = sums / jnp.maximum(cnt, 1.0)
    o_ref[...] = jnp.dot(pool, wp_ref[...],
                         preferred_element_type=jnp.float32) + bp_ref[...]


_fin_call = pl.pallas_call(
    _fin_body, out_shape=jax.ShapeDtypeStruct((G, OUTC), jnp.float32))


def kernel(edges, features, label_masks, batch, W1, b1, W2, b2, Wp, bp):
    src = edges[0]
    dst = edges[1]
    pad = EPAD - E
    src_p = jnp.concatenate([src, jnp.zeros((pad,), jnp.int32)])
    dst_p = jnp.concatenate([dst, jnp.full((pad,), N, jnp.int32)])
    src_f = src_p.reshape(NW, EPW)
    dst_t = dst_p.reshape(NW, NCHUNK, CHUNK)

    zrows = jnp.zeros((ZROWS, D), jnp.float32)

    x0 = _mask_call(features, label_masks)
    agg0 = _edge_agg_call(x0, src_f, dst_t, zrows)
    x1 = _x1_call(x0, agg0, W1, b1.reshape(1, D))
    agg1 = _edge_agg_call(x1, src_f, dst_t, zrows)
    out = _fin_call(x1, agg1, W2, b2.reshape(1, D), Wp, bp.reshape(1, 1),
                    batch.reshape(1, N))
    return out
